# baseline (speedup 1.0000x reference)
import functools
import jax
import jax.numpy as jnp
import numpy as np
from jax import lax
from jax.experimental import pallas as pl
from jax.experimental.pallas import tpu as pltpu
from jax.experimental.pallas import tpu_sc as plsc

_N = 10000
_E = 320000
_DIN = 512
_D = 128
_H = 8
_HD = 16
_DH = 512

_RB = 1000
_EB = 4000

_NEG = -3.0e38


def _sel_mat():
    s = np.zeros((_D, _H), np.float32)
    for h in range(_H):
        s[h * _HD:(h + 1) * _HD, h] = 1.0
    return jnp.asarray(s)


def _pre_body(feats, wred, bred, wq, bq, wk, bk, wv, bv, x_o, q_o, k_o, v_o):
    x = jnp.dot(feats[...], wred[...], preferred_element_type=jnp.float32) + bred[...]
    x_o[...] = x
    q_o[...] = jnp.dot(x, wq[...], preferred_element_type=jnp.float32) + bq[...]
    k_o[...] = jnp.dot(x, wk[...], preferred_element_type=jnp.float32) + bk[...]
    v_o[...] = jnp.dot(x, wv[...], preferred_element_type=jnp.float32) + bv[...]


def _pre_call(feats, wred, bred, wq, bq, wk, bk, wv, bv):
    nsteps = _N // _RB
    full = lambda shape: pl.BlockSpec(shape, lambda i: (0, 0))
    row = lambda width: pl.BlockSpec((_RB, width), lambda i: (i, 0))
    return pl.pallas_call(
        _pre_body,
        grid=(nsteps,),
        in_specs=[row(_DIN), full((_DIN, _D)), full((1, _D)),
                  full((_D, _D)), full((1, _D)), full((_D, _D)), full((1, _D)),
                  full((_D, _D)), full((1, _D))],
        out_specs=[row(_D), row(_D), row(_D), row(_D)],
        out_shape=[jax.ShapeDtypeStruct((_N, _D), jnp.float32)] * 4,
    )(feats, wred, bred, wq, bq, wk, bk, wv, bv)


def _qkv_body(x, wq, bq, wk, bk, wv, bv, q_o, k_o, v_o):
    x = x[...]
    q_o[...] = jnp.dot(x, wq[...], preferred_element_type=jnp.float32) + bq[...]
    k_o[...] = jnp.dot(x, wk[...], preferred_element_type=jnp.float32) + bk[...]
    v_o[...] = jnp.dot(x, wv[...], preferred_element_type=jnp.float32) + bv[...]


def _qkv_call(x, wq, bq, wk, bk, wv, bv):
    nsteps = _N // _RB
    full = lambda shape: pl.BlockSpec(shape, lambda i: (0, 0))
    row = lambda width: pl.BlockSpec((_RB, width), lambda i: (i, 0))
    return pl.pallas_call(
        _qkv_body,
        grid=(nsteps,),
        in_specs=[row(_D), full((_D, _D)), full((1, _D)), full((_D, _D)),
                  full((1, _D)), full((_D, _D)), full((1, _D))],
        out_specs=[row(_D), row(_D), row(_D)],
        out_shape=[jax.ShapeDtypeStruct((_N, _D), jnp.float32)] * 3,
    )(x, wq, bq, wk, bk, wv, bv)


def _score_body(qc, kr, sel, s_o, m_o):
    i = pl.program_id(0)
    s = jnp.dot(qc[...] * kr[...], sel[...],
                preferred_element_type=jnp.float32) * (1.0 / 4.0)
    s_o[...] = s
    mb = jnp.max(s, axis=0, keepdims=True)

    @pl.when(i == 0)
    def _():
        m_o[...] = mb

    @pl.when(i > 0)
    def _():
        m_o[...] = jnp.maximum(m_o[...], mb)


def _score_call(qc, kr, sel):
    nsteps = _E // _EB
    return pl.pallas_call(
        _score_body,
        grid=(nsteps,),
        in_specs=[pl.BlockSpec((_EB, _D), lambda i: (i, 0)),
                  pl.BlockSpec((_EB, _D), lambda i: (i, 0)),
                  pl.BlockSpec((_D, _H), lambda i: (0, 0))],
        out_specs=[pl.BlockSpec((_EB, _H), lambda i: (i, 0)),
                   pl.BlockSpec((1, _H), lambda i: (0, 0))],
        out_shape=[jax.ShapeDtypeStruct((_E, _H), jnp.float32),
                   jax.ShapeDtypeStruct((1, _H), jnp.float32)],
    )(qc, kr, sel)


def _max_body(s, m_o):
    i = pl.program_id(0)
    mb = jnp.max(s[...], axis=0, keepdims=True)

    @pl.when(i == 0)
    def _():
        m_o[...] = mb

    @pl.when(i > 0)
    def _():
        m_o[...] = jnp.maximum(m_o[...], mb)


def _max_call(s):
    nsteps = _E // _EB
    return pl.pallas_call(
        _max_body,
        grid=(nsteps,),
        in_specs=[pl.BlockSpec((_EB, _H), lambda i: (i, 0))],
        out_specs=[pl.BlockSpec((1, _H), lambda i: (0, 0))],
        out_shape=[jax.ShapeDtypeStruct((1, _H), jnp.float32)],
    )(s)[0]


def _ex_body(s, m, ex_o):
    ex = jnp.exp(s[...] - m[...])
    ex_o[...] = jnp.concatenate([ex, jnp.zeros_like(ex)], axis=-1)


def _ex_call(s, m):
    nsteps = _E // _EB
    return pl.pallas_call(
        _ex_body,
        grid=(nsteps,),
        in_specs=[pl.BlockSpec((_EB, _H), lambda i: (i, 0)),
                  pl.BlockSpec((1, _H), lambda i: (0, 0))],
        out_specs=[pl.BlockSpec((_EB, 2 * _H), lambda i: (i, 0))],
        out_shape=[jax.ShapeDtypeStruct((_E, 2 * _H), jnp.float32)],
    )(s, m)[0]


def _ln(x, w, b):
    mu = jnp.mean(x, axis=-1, keepdims=True)
    xc = x - mu
    var = jnp.mean(xc * xc, axis=-1, keepdims=True)
    return xc * jax.lax.rsqrt(var + 1e-5) * w + b


def _post_body(x, agg0, agg1, den0, den1, wo, bo, w1, b1, w2, b2,
               l1w, l1b, l2w, l2b, x_o):
    agg = (agg0[...] + agg1[...]) / (den0[...] + den1[...] + 1e-16)
    f2 = jnp.dot(agg, wo[...], preferred_element_type=jnp.float32) + bo[...]
    y = _ln(x[...] + f2, l1w[...], l1b[...])
    z = jnp.dot(jnp.maximum(jnp.dot(y, w1[...], preferred_element_type=jnp.float32)
                            + b1[...], 0.0),
                w2[...], preferred_element_type=jnp.float32) + b2[...]
    x_o[...] = _ln(y + z, l2w[...], l2b[...])


def _post_call(x, agg0, agg1, den0, den1, wo, bo, w1, b1, w2, b2,
               l1w, l1b, l2w, l2b):
    nsteps = _N // _RB
    full = lambda shape: pl.BlockSpec(shape, lambda i: (0, 0))
    row = lambda width: pl.BlockSpec((_RB, width), lambda i: (i, 0))
    return pl.pallas_call(
        _post_body,
        grid=(nsteps,),
        in_specs=[row(_D), row(_D), row(_D), row(_D), row(_D),
                  full((_D, _D)), full((1, _D)),
                  full((_D, _DH)), full((1, _DH)), full((_DH, _D)), full((1, _D)),
                  full((1, _D)), full((1, _D)), full((1, _D)), full((1, _D))],
        out_specs=[row(_D)],
        out_shape=[jax.ShapeDtypeStruct((_N, _D), jnp.float32)],
    )(x, agg0, agg1, den0, den1, wo, bo, w1, b1, w2, b2,
      l1w, l1b, l2w, l2b)[0]


_NC = 2
_NS = 16
_NW = _NC * _NS
_CH = 80
_EW = _E // _NW
_NCH = _EW // _CH
_NP = 10240
_NR = _NP // _NS


_RING = 5


def _gather2_build():
    mesh = plsc.VectorSubcoreMesh(core_axis_name="c", subcore_axis_name="s")

    @functools.partial(
        pl.kernel, mesh=mesh,
        out_type=jax.ShapeDtypeStruct((_E * _H,), jnp.float32),
        scratch_types=[
            pltpu.VMEM((_RING, _CH), jnp.int32),
            pltpu.VMEM((_RING, _CH), jnp.int32),
            pltpu.VMEM((_RING, _CH, _D), jnp.float32),
            pltpu.VMEM((_RING, _CH, _D), jnp.float32),
            pltpu.VMEM((_RING, 1, _CH * _H), jnp.float32),
            pltpu.SemaphoreType.DMA((_RING,)),
            pltpu.SemaphoreType.DMA((_RING,)),
            pltpu.SemaphoreType.DMA((_RING,)),
            pltpu.SemaphoreType.DMA((_RING,)),
            pltpu.SemaphoreType.DMA((_RING,)),
        ],
        compiler_params=pltpu.CompilerParams(needs_layout_passes=False),
    )
    def gat(cidx_h, ridx_h, q_h, k_h, s_o,
            cvr, rvr, qb, kb, sb, sq, sk, ws, sci, sri):
        wid = lax.axis_index("s") * _NC + lax.axis_index("c")
        base = wid * _EW

        def g_ci(j, b):
            return pltpu.make_async_copy(cidx_h.at[pl.ds(base + j * _CH, _CH)],
                                         cvr.at[b], sci.at[b])

        def g_ri(j, b):
            return pltpu.make_async_copy(ridx_h.at[pl.ds(base + j * _CH, _CH)],
                                         rvr.at[b], sri.at[b])

        def g_q(b):
            return pltpu.make_async_copy(q_h.at[cvr.at[b]], qb.at[b], sq.at[b])

        def g_k(b):
            return pltpu.make_async_copy(k_h.at[rvr.at[b]], kb.at[b], sk.at[b])

        def w_s(j, b):
            off = (base + j * _CH) * _H
            return pltpu.make_async_copy(sb.at[b, 0],
                                         s_o.at[pl.ds(off, _CH * _H)],
                                         ws.at[b])

        def issue_idx(j, b):
            g_ci(j, b).start()
            g_ri(j, b).start()

        def wait_idx(j, b):
            g_ci(j, b).wait()
            g_ri(j, b).wait()

        issue_idx(0, 0)
        issue_idx(1, 1)
        wait_idx(0, 0)
        g_q(0).start()
        g_k(0).start()

        def body(g, carry):
            for b in range(_RING):
                j = g * _RING + b
                nb = (b + 1) % _RING

                @pl.when(j >= _RING - 1)
                def _():
                    w_s(j - (_RING - 1), nb).wait()

                @pl.when(j + 2 < _NCH)
                def _():
                    issue_idx(j + 2, (b + 2) % _RING)

                @pl.when(j + 1 < _NCH)
                def _():
                    wait_idx(j + 1, nb)
                    g_q(nb).start()
                    g_k(nb).start()

                g_q(b).wait()
                g_k(b).wait()
                fb = jnp.full((16,), b, jnp.int32)
                fz = jnp.zeros((16,), jnp.int32)
                lanes = lax.iota(jnp.int32, 16)
                m0 = lanes == 0

                def dot4(e0, carry2):
                    for u in range(4):
                        e = e0 * 4 + u
                        fe = jnp.full((16,), e, jnp.int32)
                        for h in range(_H):
                            ln = lanes + h * _HD
                            lq = plsc.load_gather(qb, [fb, fe, ln])
                            lk = plsc.load_gather(kb, [fb, fe, ln])
                            sc = jnp.sum(lq * lk) * 0.25
                            plsc.store_scatter(
                                sb, [fb, fz, jnp.full((16,), e * _H + h,
                                                      jnp.int32)],
                                jnp.full((16,), sc, jnp.float32), mask=m0)
                    return carry2

                lax.fori_loop(0, _CH // 4, dot4, 0)
                w_s(j, b).start()
            return carry

        lax.fori_loop(0, _NCH // _RING, body, 0)
        for j in range(_NCH - _RING + 1, _NCH):
            w_s(j, j % _RING).wait()

    return gat


_ES = _E // _NS
_CHF = 40
_NCHW = _EW // _CHF
_CIR = 10


def _fuse2_build():
    mesh = plsc.VectorSubcoreMesh(core_axis_name="c", subcore_axis_name="s")

    @functools.partial(
        pl.kernel, mesh=mesh,
        out_type=[jax.ShapeDtypeStruct((_NC, _NP, _D), jnp.float32),
                  jax.ShapeDtypeStruct((_NC, _NP, _D), jnp.float32)],
        scratch_types=[
            pltpu.VMEM((_CIR, _CHF), jnp.int32),
            pltpu.VMEM((_RING, _CHF), jnp.int32),
            pltpu.VMEM((_RING, _CHF, _D), jnp.float32),
            pltpu.VMEM((_RING, 1, _CHF * 2 * _H), jnp.float32),
            pltpu.VMEM_SHARED((_NP, _D), jnp.float32),
            pltpu.SemaphoreType.DMA((_RING,)),
            pltpu.SemaphoreType.DMA((_RING,)),
            pltpu.SemaphoreType.DMA((_RING,)),
            pltpu.SemaphoreType.DMA((_CIR,)),
            pltpu.SemaphoreType.DMA((_RING,)),
        ],
        compiler_params=pltpu.CompilerParams(needs_layout_passes=False),
    )
    def fus(cidx_h, ridx_h, v_h, ex_h, zero_h, agg_o, den_o,
            cvr, rvr, wb, eb, acc_sh, sv, se, ss, sci, sri):
        cid = lax.axis_index("c")
        sid = lax.axis_index("s")
        rb = sid * _NR
        wid = sid * _NC + cid
        base = wid * _EW
        pltpu.sync_copy(zero_h.at[pl.ds(rb, _NR)], acc_sh.at[pl.ds(rb, _NR)])
        plsc.subcore_barrier()

        def g_ci(j, bc):
            return pltpu.make_async_copy(
                cidx_h.at[pl.ds(base + j * _CHF, _CHF)], cvr.at[bc],
                sci.at[bc])

        def g_ri(j, b):
            return pltpu.make_async_copy(
                ridx_h.at[pl.ds(base + j * _CHF, _CHF)], rvr.at[b],
                sri.at[b])

        def g_v(b):
            return pltpu.make_async_copy(v_h.at[rvr.at[b]], wb.at[b],
                                         sv.at[b])

        def g_e(j, b):
            off = (base + j * _CHF) * 2 * _H
            return pltpu.make_async_copy(ex_h.at[pl.ds(off, _CHF * 2 * _H)],
                                         eb.at[b, 0], se.at[b])

        def s_a(b, bc):
            return pltpu.make_async_copy(wb.at[b], acc_sh.at[cvr.at[bc]],
                                         ss.at[b])

        def run_phase(with_v):
            def issue_idx(j, bc, br):
                g_ci(j, bc).start()
                if with_v:
                    g_ri(j, br).start()

            def wait_idx(j, bc, br):
                g_ci(j, bc).wait()
                if with_v:
                    g_ri(j, br).wait()

            def issue_data(j, b):
                g_e(j, b).start()
                if with_v:
                    g_v(b).start()

            issue_idx(0, 0, 0)
            issue_idx(1, 1, 1)
            wait_idx(0, 0, 0)
            issue_data(0, 0)

            def body(g, carry):
                for bb in range(_CIR):
                    j = g * _CIR + bb
                    b = bb % _RING
                    nb = (b + 1) % _RING

                    @pl.when(j >= 4)
                    def _():
                        s_a(nb, (bb + 6) % _CIR).wait()

                    @pl.when(j + 2 < _NCHW)
                    def _():
                        issue_idx(j + 2, (bb + 2) % _CIR, (bb + 2) % _RING)

                    @pl.when(j + 1 < _NCHW)
                    def _():
                        wait_idx(j + 1, (bb + 1) % _CIR, nb)
                        issue_data(j + 1, nb)

                    g_e(j, b).wait()
                    fb = jnp.full((16,), b, jnp.int32)
                    fz = jnp.zeros((16,), jnp.int32)
                    lanes = lax.iota(jnp.int32, 16)

                    if with_v:
                        g_v(b).wait()

                        def mul4(e0, carry2):
                            for u in range(4):
                                e = e0 * 4 + u
                                fe = jnp.full((16,), e, jnp.int32)
                                exv = plsc.load_gather(
                                    eb, [fb, fz, lanes + e * (2 * _H)])
                                for h in range(_H):
                                    ln = lanes + h * _HD
                                    cur = plsc.load_gather(wb, [fb, fe, ln])
                                    plsc.store_scatter(wb, [fb, fe, ln],
                                                       cur * exv[h])
                            return carry2

                        lax.fori_loop(0, _CHF // 4, mul4, 0)
                    else:
                        def rep4(e0, carry2):
                            for u in range(4):
                                e = e0 * 4 + u
                                fe = jnp.full((16,), e, jnp.int32)
                                exv = plsc.load_gather(
                                    eb, [fb, fz, lanes + e * (2 * _H)])
                                for h in range(_H):
                                    plsc.store_scatter(
                                        wb, [fb, fe, lanes + h * _HD],
                                        jnp.full((_HD,), exv[h], jnp.float32))
                            return carry2

                        lax.fori_loop(0, _CHF // 4, rep4, 0)

                    s_a(b, bb).start(add=True)
                return carry

            lax.fori_loop(0, _NCHW // _CIR, body, 0)
            for j in range(_NCHW - 4, _NCHW):
                s_a(j % _RING, j % _CIR).wait()

        run_phase(True)
        plsc.subcore_barrier()
        pltpu.sync_copy(acc_sh.at[pl.ds(rb, _NR)],
                        agg_o.at[cid, pl.ds(rb, _NR)])
        pltpu.sync_copy(zero_h.at[pl.ds(rb, _NR)], acc_sh.at[pl.ds(rb, _NR)])
        plsc.subcore_barrier()
        run_phase(False)
        plsc.subcore_barrier()
        pltpu.sync_copy(acc_sh.at[pl.ds(rb, _NR)],
                        den_o.at[cid, pl.ds(rb, _NR)])

    return fus


_g2 = _gather2_build()
_f2 = _fuse2_build()


def kernel(feats, edge_index, edge_attr, W_red, b_red,
           Wq0, Wk0, Wv0, Wo0, bq0, bk0, bv0, bo0,
           W10, b10, W20, b20, ln1w0, ln1b0, ln2w0, ln2b0,
           Wq1, Wk1, Wv1, Wo1, bq1, bk1, bv1, bo1,
           W11, b11, W21, b21, ln1w1, ln1b1, ln2w1, ln2b1):
    sel = _sel_mat()
    selt = sel.T
    r1 = edge_index[:, 0]
    c1 = edge_index[:, 1]
    zacc = jnp.zeros((_NP, _D), jnp.float32)
    v2 = lambda a: a.reshape(1, -1)

    x, q, k, v = _pre_call(feats, W_red, v2(b_red),
                           Wq0, v2(bq0), Wk0, v2(bk0), Wv0, v2(bv0))

    layers = [
        (Wq0, bq0, Wk0, bk0, Wv0, bv0, Wo0, bo0, W10, b10, W20, b20,
         ln1w0, ln1b0, ln2w0, ln2b0),
        (Wq1, bq1, Wk1, bk1, Wv1, bv1, Wo1, bo1, W11, b11, W21, b21,
         ln1w1, ln1b1, ln2w1, ln2b1),
    ]
    for li, (wq, bq, wk, bk, wv, bv, wo, bo, w1, b1, w2, b2,
             l1w, l1b, l2w, l2b) in enumerate(layers):
        if li > 0:
            q, k, v = _qkv_call(x, wq, v2(bq), wk, v2(bk), wv, v2(bv))
        s = _g2(c1, r1, q, k).reshape(_E, _H)
        m = _max_call(s)
        ex = _ex_call(s, m).reshape(_E * 2 * _H)
        agg_p, den_p = _f2(c1, r1, v, ex, zacc)
        x = _post_call(x, agg_p[0, :_N], agg_p[1, :_N],
                       den_p[0, :_N], den_p[1, :_N], wo, v2(bo),
                       w1, v2(b1), w2, v2(b2), v2(l1w), v2(l1b),
                       v2(l2w), v2(l2b))
    return x


# speedup vs baseline: 2.2568x; 2.2568x over previous
import functools
import jax
import jax.numpy as jnp
import numpy as np
from jax import lax
from jax.experimental import pallas as pl
from jax.experimental.pallas import tpu as pltpu
from jax.experimental.pallas import tpu_sc as plsc

_N = 10000
_E = 320000
_DIN = 512
_D = 128
_H = 8
_HD = 16
_DH = 512

_RB = 1000
_EB = 4000

_NEG = -3.0e38


def _sel_mat():
    s = np.zeros((_D, _H), np.float32)
    for h in range(_H):
        s[h * _HD:(h + 1) * _HD, h] = 1.0
    return jnp.asarray(s)


def _pre_body(feats, wred, bred, wq, bq, wk, bk, wv, bv, x_o, q_o, k_o, v_o):
    x = jnp.dot(feats[...], wred[...], preferred_element_type=jnp.float32) + bred[...]
    x_o[...] = x
    q_o[...] = jnp.dot(x, wq[...], preferred_element_type=jnp.float32) + bq[...]
    k_o[...] = jnp.dot(x, wk[...], preferred_element_type=jnp.float32) + bk[...]
    v_o[...] = jnp.dot(x, wv[...], preferred_element_type=jnp.float32) + bv[...]


def _pre_call(feats, wred, bred, wq, bq, wk, bk, wv, bv):
    nsteps = _N // _RB
    full = lambda shape: pl.BlockSpec(shape, lambda i: (0, 0))
    row = lambda width: pl.BlockSpec((_RB, width), lambda i: (i, 0))
    return pl.pallas_call(
        _pre_body,
        grid=(nsteps,),
        in_specs=[row(_DIN), full((_DIN, _D)), full((1, _D)),
                  full((_D, _D)), full((1, _D)), full((_D, _D)), full((1, _D)),
                  full((_D, _D)), full((1, _D))],
        out_specs=[row(_D), row(_D), row(_D), row(_D)],
        out_shape=[jax.ShapeDtypeStruct((_N, _D), jnp.float32)] * 4,
    )(feats, wred, bred, wq, bq, wk, bk, wv, bv)


def _qkv_body(x, wq, bq, wk, bk, wv, bv, q_o, k_o, v_o):
    x = x[...]
    q_o[...] = jnp.dot(x, wq[...], preferred_element_type=jnp.float32) + bq[...]
    k_o[...] = jnp.dot(x, wk[...], preferred_element_type=jnp.float32) + bk[...]
    v_o[...] = jnp.dot(x, wv[...], preferred_element_type=jnp.float32) + bv[...]


def _qkv_call(x, wq, bq, wk, bk, wv, bv):
    nsteps = _N // _RB
    full = lambda shape: pl.BlockSpec(shape, lambda i: (0, 0))
    row = lambda width: pl.BlockSpec((_RB, width), lambda i: (i, 0))
    return pl.pallas_call(
        _qkv_body,
        grid=(nsteps,),
        in_specs=[row(_D), full((_D, _D)), full((1, _D)), full((_D, _D)),
                  full((1, _D)), full((_D, _D)), full((1, _D))],
        out_specs=[row(_D), row(_D), row(_D)],
        out_shape=[jax.ShapeDtypeStruct((_N, _D), jnp.float32)] * 3,
    )(x, wq, bq, wk, bk, wv, bv)


def _score_body(qc, kr, sel, s_o, m_o):
    i = pl.program_id(0)
    s = jnp.dot(qc[...] * kr[...], sel[...],
                preferred_element_type=jnp.float32) * (1.0 / 4.0)
    s_o[...] = s
    mb = jnp.max(s, axis=0, keepdims=True)

    @pl.when(i == 0)
    def _():
        m_o[...] = mb

    @pl.when(i > 0)
    def _():
        m_o[...] = jnp.maximum(m_o[...], mb)


def _score_call(qc, kr, sel):
    nsteps = _E // _EB
    return pl.pallas_call(
        _score_body,
        grid=(nsteps,),
        in_specs=[pl.BlockSpec((_EB, _D), lambda i: (i, 0)),
                  pl.BlockSpec((_EB, _D), lambda i: (i, 0)),
                  pl.BlockSpec((_D, _H), lambda i: (0, 0))],
        out_specs=[pl.BlockSpec((_EB, _H), lambda i: (i, 0)),
                   pl.BlockSpec((1, _H), lambda i: (0, 0))],
        out_shape=[jax.ShapeDtypeStruct((_E, _H), jnp.float32),
                   jax.ShapeDtypeStruct((1, _H), jnp.float32)],
    )(qc, kr, sel)


def _ex_body(s, m, ex_o):
    ex = jnp.exp(s[...] - m[...])
    ex_o[...] = jnp.concatenate([ex, jnp.zeros_like(ex)], axis=-1)


def _ex_call(s, m):
    nsteps = _E // _EB
    return pl.pallas_call(
        _ex_body,
        grid=(nsteps,),
        in_specs=[pl.BlockSpec((_EB, _H), lambda i: (i, 0)),
                  pl.BlockSpec((1, _H), lambda i: (0, 0))],
        out_specs=[pl.BlockSpec((_EB, 2 * _H), lambda i: (i, 0))],
        out_shape=[jax.ShapeDtypeStruct((_E, 2 * _H), jnp.float32)],
    )(s, m)[0]


def _ln(x, w, b):
    mu = jnp.mean(x, axis=-1, keepdims=True)
    xc = x - mu
    var = jnp.mean(xc * xc, axis=-1, keepdims=True)
    return xc * jax.lax.rsqrt(var + 1e-5) * w + b


def _post_body(x, agg0, agg1, den0, den1, wo, bo, w1, b1, w2, b2,
               l1w, l1b, l2w, l2b, x_o):
    agg = (agg0[...] + agg1[...]) / (den0[...] + den1[...] + 1e-16)
    f2 = jnp.dot(agg, wo[...], preferred_element_type=jnp.float32) + bo[...]
    y = _ln(x[...] + f2, l1w[...], l1b[...])
    z = jnp.dot(jnp.maximum(jnp.dot(y, w1[...], preferred_element_type=jnp.float32)
                            + b1[...], 0.0),
                w2[...], preferred_element_type=jnp.float32) + b2[...]
    x_o[...] = _ln(y + z, l2w[...], l2b[...])


def _post_call(x, agg0, agg1, den0, den1, wo, bo, w1, b1, w2, b2,
               l1w, l1b, l2w, l2b):
    nsteps = _N // _RB
    full = lambda shape: pl.BlockSpec(shape, lambda i: (0, 0))
    row = lambda width: pl.BlockSpec((_RB, width), lambda i: (i, 0))
    return pl.pallas_call(
        _post_body,
        grid=(nsteps,),
        in_specs=[row(_D), row(_D), row(_D), row(_D), row(_D),
                  full((_D, _D)), full((1, _D)),
                  full((_D, _DH)), full((1, _DH)), full((_DH, _D)), full((1, _D)),
                  full((1, _D)), full((1, _D)), full((1, _D)), full((1, _D))],
        out_specs=[row(_D)],
        out_shape=[jax.ShapeDtypeStruct((_N, _D), jnp.float32)],
    )(x, agg0, agg1, den0, den1, wo, bo, w1, b1, w2, b2,
      l1w, l1b, l2w, l2b)[0]


_NC = 2
_NS = 16
_NW = _NC * _NS
_CH = 80
_EW = _E // _NW
_NCH = _EW // _CH
_NP = 10240
_NR = _NP // _NS


_RING = 5


def _gather2_build():
    mesh = plsc.VectorSubcoreMesh(core_axis_name="c", subcore_axis_name="s")

    @functools.partial(
        pl.kernel, mesh=mesh,
        out_type=[jax.ShapeDtypeStruct((_E, _D), jnp.float32)] * 2,
        scratch_types=[
            pltpu.VMEM((_RING, _CH), jnp.int32),
            pltpu.VMEM((_RING, _CH), jnp.int32),
            pltpu.VMEM((_RING, _CH, _D), jnp.float32),
            pltpu.VMEM((_RING, _CH, _D), jnp.float32),
            pltpu.SemaphoreType.DMA((_RING,)),
            pltpu.SemaphoreType.DMA((_RING,)),
            pltpu.SemaphoreType.DMA((_RING,)),
            pltpu.SemaphoreType.DMA((_RING,)),
            pltpu.SemaphoreType.DMA((_RING,)),
            pltpu.SemaphoreType.DMA((_RING,)),
        ],
    )
    def gat(cidx_h, ridx_h, q_h, k_h, qc_o, kr_o,
            cvr, rvr, qb, kb, sq, sk, wq, wk, sci, sri):
        wid = lax.axis_index("s") * _NC + lax.axis_index("c")
        base = wid * _EW

        def g_ci(j, b):
            return pltpu.make_async_copy(cidx_h.at[pl.ds(base + j * _CH, _CH)],
                                         cvr.at[b], sci.at[b])

        def g_ri(j, b):
            return pltpu.make_async_copy(ridx_h.at[pl.ds(base + j * _CH, _CH)],
                                         rvr.at[b], sri.at[b])

        def g_q(b):
            return pltpu.make_async_copy(q_h.at[cvr.at[b]], qb.at[b], sq.at[b])

        def g_k(b):
            return pltpu.make_async_copy(k_h.at[rvr.at[b]], kb.at[b], sk.at[b])

        def w_q(j, b):
            off = base + j * _CH
            return pltpu.make_async_copy(qb.at[b], qc_o.at[pl.ds(off, _CH)],
                                         wq.at[b])

        def w_k(j, b):
            off = base + j * _CH
            return pltpu.make_async_copy(kb.at[b], kr_o.at[pl.ds(off, _CH)],
                                         wk.at[b])

        def issue_idx(j, b):
            g_ci(j, b).start()
            g_ri(j, b).start()

        def wait_idx(j, b):
            g_ci(j, b).wait()
            g_ri(j, b).wait()

        issue_idx(0, 0)
        issue_idx(1, 1)
        wait_idx(0, 0)
        g_q(0).start()
        g_k(0).start()

        def body(g, carry):
            for b in range(_RING):
                j = g * _RING + b
                nb = (b + 1) % _RING

                @pl.when(j >= _RING - 1)
                def _():
                    w_q(j - (_RING - 1), nb).wait()
                    w_k(j - (_RING - 1), nb).wait()

                @pl.when(j + 2 < _NCH)
                def _():
                    issue_idx(j + 2, (b + 2) % _RING)

                @pl.when(j + 1 < _NCH)
                def _():
                    wait_idx(j + 1, nb)
                    g_q(nb).start()
                    g_k(nb).start()

                g_q(b).wait()
                g_k(b).wait()
                w_q(j, b).start()
                w_k(j, b).start()
            return carry

        lax.fori_loop(0, _NCH // _RING, body, 0)
        for j in range(_NCH - _RING + 1, _NCH):
            w_q(j, j % _RING).wait()
            w_k(j, j % _RING).wait()

    return gat


_ES = _E // _NS
_CHF = 40
_NCHW = _EW // _CHF
_CIR = 10


def _fuse2_build():
    mesh = plsc.VectorSubcoreMesh(core_axis_name="c", subcore_axis_name="s")

    @functools.partial(
        pl.kernel, mesh=mesh,
        out_type=[jax.ShapeDtypeStruct((_NC, _NP, _D), jnp.float32),
                  jax.ShapeDtypeStruct((_NC, _NP, _D), jnp.float32)],
        scratch_types=[
            pltpu.VMEM((_CIR, _CHF), jnp.int32),
            pltpu.VMEM((_RING, _CHF), jnp.int32),
            pltpu.VMEM((_RING, _CHF, _D), jnp.float32),
            pltpu.VMEM((_RING, 1, _CHF * 2 * _H), jnp.float32),
            pltpu.VMEM_SHARED((_NP, _D), jnp.float32),
            pltpu.SemaphoreType.DMA((_RING,)),
            pltpu.SemaphoreType.DMA((_RING,)),
            pltpu.SemaphoreType.DMA((_RING,)),
            pltpu.SemaphoreType.DMA((_CIR,)),
            pltpu.SemaphoreType.DMA((_RING,)),
        ],
        compiler_params=pltpu.CompilerParams(needs_layout_passes=False),
    )
    def fus(cidx_h, ridx_h, v_h, ex_h, zero_h, agg_o, den_o,
            cvr, rvr, wb, eb, acc_sh, sv, se, ss, sci, sri):
        cid = lax.axis_index("c")
        sid = lax.axis_index("s")
        rb = sid * _NR
        wid = sid * _NC + cid
        base = wid * _EW
        pltpu.sync_copy(zero_h.at[pl.ds(rb, _NR)], acc_sh.at[pl.ds(rb, _NR)])
        plsc.subcore_barrier()

        def g_ci(j, bc):
            return pltpu.make_async_copy(
                cidx_h.at[pl.ds(base + j * _CHF, _CHF)], cvr.at[bc],
                sci.at[bc])

        def g_ri(j, b):
            return pltpu.make_async_copy(
                ridx_h.at[pl.ds(base + j * _CHF, _CHF)], rvr.at[b],
                sri.at[b])

        def g_v(b):
            return pltpu.make_async_copy(v_h.at[rvr.at[b]], wb.at[b],
                                         sv.at[b])

        def g_e(j, b):
            off = (base + j * _CHF) * 2 * _H
            return pltpu.make_async_copy(ex_h.at[pl.ds(off, _CHF * 2 * _H)],
                                         eb.at[b, 0], se.at[b])

        def s_a(b, bc):
            return pltpu.make_async_copy(wb.at[b], acc_sh.at[cvr.at[bc]],
                                         ss.at[b])

        def run_phase(with_v):
            def issue_idx(j, bc, br):
                g_ci(j, bc).start()
                if with_v:
                    g_ri(j, br).start()

            def wait_idx(j, bc, br):
                g_ci(j, bc).wait()
                if with_v:
                    g_ri(j, br).wait()

            def issue_data(j, b):
                g_e(j, b).start()
                if with_v:
                    g_v(b).start()

            issue_idx(0, 0, 0)
            issue_idx(1, 1, 1)
            wait_idx(0, 0, 0)
            issue_data(0, 0)

            def body(g, carry):
                for bb in range(_CIR):
                    j = g * _CIR + bb
                    b = bb % _RING
                    nb = (b + 1) % _RING

                    @pl.when(j >= 4)
                    def _():
                        s_a(nb, (bb + 6) % _CIR).wait()

                    @pl.when(j + 2 < _NCHW)
                    def _():
                        issue_idx(j + 2, (bb + 2) % _CIR, (bb + 2) % _RING)

                    @pl.when(j + 1 < _NCHW)
                    def _():
                        wait_idx(j + 1, (bb + 1) % _CIR, nb)
                        issue_data(j + 1, nb)

                    g_e(j, b).wait()
                    fb = jnp.full((16,), b, jnp.int32)
                    fz = jnp.zeros((16,), jnp.int32)
                    lanes = lax.iota(jnp.int32, 16)

                    if with_v:
                        g_v(b).wait()

                        def mul4(e0, carry2):
                            for u in range(4):
                                e = e0 * 4 + u
                                fe = jnp.full((16,), e, jnp.int32)
                                exv = plsc.load_gather(
                                    eb, [fb, fz, lanes + e * (2 * _H)])
                                for h in range(_H):
                                    ln = lanes + h * _HD
                                    cur = plsc.load_gather(wb, [fb, fe, ln])
                                    plsc.store_scatter(wb, [fb, fe, ln],
                                                       cur * exv[h])
                            return carry2

                        lax.fori_loop(0, _CHF // 4, mul4, 0)
                    else:
                        def rep4(e0, carry2):
                            for u in range(4):
                                e = e0 * 4 + u
                                fe = jnp.full((16,), e, jnp.int32)
                                exv = plsc.load_gather(
                                    eb, [fb, fz, lanes + e * (2 * _H)])
                                for h in range(_H):
                                    plsc.store_scatter(
                                        wb, [fb, fe, lanes + h * _HD],
                                        jnp.full((_HD,), exv[h], jnp.float32))
                            return carry2

                        lax.fori_loop(0, _CHF // 4, rep4, 0)

                    s_a(b, bb).start(add=True)
                return carry

            lax.fori_loop(0, _NCHW // _CIR, body, 0)
            for j in range(_NCHW - 4, _NCHW):
                s_a(j % _RING, j % _CIR).wait()

        run_phase(True)
        plsc.subcore_barrier()
        pltpu.sync_copy(acc_sh.at[pl.ds(rb, _NR)],
                        agg_o.at[cid, pl.ds(rb, _NR)])
        pltpu.sync_copy(zero_h.at[pl.ds(rb, _NR)], acc_sh.at[pl.ds(rb, _NR)])
        plsc.subcore_barrier()
        run_phase(False)
        plsc.subcore_barrier()
        pltpu.sync_copy(acc_sh.at[pl.ds(rb, _NR)],
                        den_o.at[cid, pl.ds(rb, _NR)])

    return fus


_g2 = _gather2_build()
_f2 = _fuse2_build()


def kernel(feats, edge_index, edge_attr, W_red, b_red,
           Wq0, Wk0, Wv0, Wo0, bq0, bk0, bv0, bo0,
           W10, b10, W20, b20, ln1w0, ln1b0, ln2w0, ln2b0,
           Wq1, Wk1, Wv1, Wo1, bq1, bk1, bv1, bo1,
           W11, b11, W21, b21, ln1w1, ln1b1, ln2w1, ln2b1):
    sel = _sel_mat()
    selt = sel.T
    r1 = edge_index[:, 0]
    c1 = edge_index[:, 1]
    zacc = jnp.zeros((_NP, _D), jnp.float32)
    v2 = lambda a: a.reshape(1, -1)

    x, q, k, v = _pre_call(feats, W_red, v2(b_red),
                           Wq0, v2(bq0), Wk0, v2(bk0), Wv0, v2(bv0))

    layers = [
        (Wq0, bq0, Wk0, bk0, Wv0, bv0, Wo0, bo0, W10, b10, W20, b20,
         ln1w0, ln1b0, ln2w0, ln2b0),
        (Wq1, bq1, Wk1, bk1, Wv1, bv1, Wo1, bo1, W11, b11, W21, b21,
         ln1w1, ln1b1, ln2w1, ln2b1),
    ]
    for li, (wq, bq, wk, bk, wv, bv, wo, bo, w1, b1, w2, b2,
             l1w, l1b, l2w, l2b) in enumerate(layers):
        if li > 0:
            q, k, v = _qkv_call(x, wq, v2(bq), wk, v2(bk), wv, v2(bv))
        qc, kr = _g2(c1, r1, q, k)
        s, m = _score_call(qc, kr, sel)
        ex = _ex_call(s, m).reshape(_E * 2 * _H)
        agg_p, den_p = _f2(c1, r1, v, ex, zacc)
        x = _post_call(x, agg_p[0, :_N], agg_p[1, :_N],
                       den_p[0, :_N], den_p[1, :_N], wo, v2(bo),
                       w1, v2(b1), w2, v2(b2), v2(l1w), v2(l1b),
                       v2(l2w), v2(l2b))
    return x
